# baseline (speedup 1.0000x reference)
import jax
import jax.numpy as jnp
from jax.experimental import pallas as pl

_B, _N, _M = 16, 2048, 2048


def _parts(x, n):
    out = []
    for _ in range(n):
        hi = x.astype(jnp.bfloat16).astype(jnp.float32)
        out.append(hi)
        x = x - hi
    return out


def _aug(c3, extra_rows):
    return jnp.concatenate([c3] + extra_rows, axis=0)


def _one_batch(s3, t3):
    x2 = jnp.sum(s3 * s3, axis=0, keepdims=True)
    y2 = jnp.sum(t3 * t3, axis=0, keepdims=True)
    x2h, x2m, x2l = _parts(x2, 3)
    y2h, y2l = _parts(y2, 2)
    s_aug = _aug(-2.0 * s3,
                 [x2h, x2m, x2l, jnp.ones((2, _N), jnp.float32)])
    t_aug = _aug(t3,
                 [jnp.ones((3, _M), jnp.float32), y2h, y2l])
    d2 = jax.lax.dot_general(
        s_aug, t_aug, (((0,), (0,)), ((), ())),
        precision=jax.lax.Precision.DEFAULT,
        preferred_element_type=jnp.float32)
    row_total = jnp.sum(jnp.min(d2, axis=1))
    col_total = jnp.sum(jnp.min(d2, axis=0))
    return row_total / _N + col_total / _M


_BPS = 16


def _chamfer_body(srcT_ref, tgtT_ref, out_ref):
    res = jnp.float32(0.0)
    for j in range(_BPS):
        res = res + _one_batch(srcT_ref[j], tgtT_ref[j])

    @pl.when(pl.program_id(0) == 0)
    def _init():
        out_ref[...] = jnp.zeros((1, 1), jnp.float32)

    out_ref[...] = out_ref[...] + jnp.reshape(res, (1, 1))


def kernel(src, tgt):
    srcT = jnp.transpose(src, (0, 2, 1))
    tgtT = jnp.transpose(tgt, (0, 2, 1))
    total = pl.pallas_call(
        _chamfer_body,
        grid=(_B // _BPS,),
        in_specs=[
            pl.BlockSpec((_BPS, 3, _N), lambda b: (b, 0, 0)),
            pl.BlockSpec((_BPS, 3, _M), lambda b: (b, 0, 0)),
        ],
        out_specs=pl.BlockSpec((1, 1), lambda b: (0, 0)),
        out_shape=jax.ShapeDtypeStruct((1, 1), jnp.float32),
    )(srcT, tgtT)
    return total[0, 0]


# speedup vs baseline: 1.1357x; 1.1357x over previous
import jax
import jax.numpy as jnp
from jax.experimental import pallas as pl

_B, _N, _M = 16, 2048, 2048


def _parts(x, n):
    out = []
    for _ in range(n):
        hi = x.astype(jnp.bfloat16).astype(jnp.float32)
        out.append(hi)
        x = x - hi
    return out


def _aug(c3, extra_rows):
    return jnp.concatenate([c3] + extra_rows, axis=0)


def _one_batch(s3, t3):
    x2 = jnp.sum(s3 * s3, axis=0, keepdims=True)
    y2 = jnp.sum(t3 * t3, axis=0, keepdims=True)
    x2h, x2m, x2l = _parts(x2, 3)
    y2h, y2l = _parts(y2, 2)
    s_aug = _aug(-2.0 * s3,
                 [x2h, x2m, x2l, jnp.ones((2, _N), jnp.float32)])
    t_aug = _aug(t3,
                 [jnp.ones((3, _M), jnp.float32), y2h, y2l])
    d2 = jax.lax.dot_general(
        s_aug, t_aug, (((0,), (0,)), ((), ())),
        precision=jax.lax.Precision.DEFAULT,
        preferred_element_type=jnp.float32)
    row_total = jnp.sum(jnp.min(d2, axis=1))
    col_total = jnp.sum(jnp.min(d2, axis=0))
    return row_total / _N + col_total / _M


_BPS = 8


def _chamfer_body(srcT_ref, tgtT_ref, out_ref):
    res = jnp.float32(0.0)
    for j in range(_BPS):
        res = res + _one_batch(srcT_ref[j], tgtT_ref[j])

    @pl.when(pl.program_id(0) == 0)
    def _init():
        out_ref[...] = jnp.zeros((1, 1), jnp.float32)

    out_ref[...] = out_ref[...] + jnp.reshape(res, (1, 1))


def kernel(src, tgt):
    srcT = jnp.transpose(src, (0, 2, 1))
    tgtT = jnp.transpose(tgt, (0, 2, 1))
    total = pl.pallas_call(
        _chamfer_body,
        grid=(_B // _BPS,),
        in_specs=[
            pl.BlockSpec((_BPS, 3, _N), lambda b: (b, 0, 0)),
            pl.BlockSpec((_BPS, 3, _M), lambda b: (b, 0, 0)),
        ],
        out_specs=pl.BlockSpec((1, 1), lambda b: (0, 0)),
        out_shape=jax.ShapeDtypeStruct((1, 1), jnp.float32),
    )(srcT, tgtT)
    return total[0, 0]
